# Initial kernel scaffold; baseline (speedup 1.0000x reference)
import jax
import jax.numpy as jnp
from jax.experimental import pallas as pl


def kernel(x, target):
    raise NotImplementedError("write your pallas kernel here")


# speedup vs baseline: 1.6683x; 1.6683x over previous
import functools
import numpy as np
import jax
import jax.numpy as jnp
from jax import lax
from jax.experimental import pallas as pl
from jax.experimental.pallas import tpu as pltpu

_SMOOTHING = 0.1
_CONF = 1.0 - _SMOOTHING
_VB = 1024


def _tc_body(x_ref, t_ref, acc_ref, *, V, Vb, fill, conf, C, nj):
    j = pl.program_id(0)
    col = j * Vb + lax.broadcasted_iota(jnp.int32, x_ref.shape, 1)
    t = t_ref[...]
    m = t != 0
    xb = x_ref[...]
    xm = jnp.where((col > 0) & (col < V) & m, xb, 0.0)
    part = -fill * jnp.sum(xm)
    xt = jnp.sum(jnp.where((col == t) & m, xb, 0.0))
    part = part + (fill - conf) * xt

    @pl.when(j == 0)
    def _init():
        acc_ref[...] = jnp.zeros_like(acc_ref)

    acc_ref[...] += part.reshape(1, 1)

    @pl.when(j == nj - 1)
    def _tail():
        cnt = C * jnp.sum(m.astype(jnp.float32))
        acc_ref[...] += cnt.reshape(1, 1)


def _tc_sum(x, t2d):
    N, V = x.shape
    fill = _SMOOTHING / (V - 2)
    C = float(fill * np.log(fill) * (V - 2) + _CONF * np.log(_CONF))
    nj = (V + _VB - 1) // _VB
    body = functools.partial(
        _tc_body, V=V, Vb=_VB, fill=fill, conf=_CONF, C=C, nj=nj)
    return pl.pallas_call(
        body,
        grid=(nj,),
        in_specs=[
            pl.BlockSpec((N, _VB), lambda j: (0, j)),
            pl.BlockSpec((N, 1), lambda j: (0, 0)),
        ],
        out_specs=pl.BlockSpec((1, 1), lambda j: (0, 0)),
        out_shape=jax.ShapeDtypeStruct((1, 1), jnp.float32),
    )(x, t2d)


def kernel(x, target):
    N, V = x.shape
    t2d = target.astype(jnp.int32).reshape(N, 1)
    acc = _tc_sum(x, t2d)
    return acc[0, 0]
